# baseline (speedup 1.0000x reference)
import functools

import jax
import jax.numpy as jnp
from jax import lax
from jax.experimental import pallas as pl
from jax.experimental.pallas import tpu as pltpu
from jax.experimental.pallas import tpu_sc as plsc

F = 26
V = 100000
D = 32
B = 4096
EPS = 1e-5

NC = 2
NS = 16
NW = NC * NS
BW = B // NW
RW = BW * F
NB = RW // 16


def _rsqrt(x):
    i = plsc.bitcast(x, jnp.int32)
    i = jnp.int32(0x5F3759DF) - (i >> 1)
    y = plsc.bitcast(i, jnp.float32)
    for _ in range(3):
        y = y * (1.5 - 0.5 * x * y * y)
    return y


@functools.partial(
    pl.kernel,
    out_type=jax.ShapeDtypeStruct((B * F, D), jnp.float32),
    mesh=plsc.VectorSubcoreMesh(
        core_axis_name="c", subcore_axis_name="s", num_cores=NC, num_subcores=NS
    ),
    compiler_params=pltpu.CompilerParams(
        use_tc_tiling_on_sc=False, needs_layout_passes=False
    ),
    scratch_types=[
        pltpu.VMEM((F, BW), jnp.int32),
        pltpu.VMEM((F, BW), jnp.int32),
        pltpu.VMEM((RW, D), jnp.float32),
        pltpu.VMEM((D,), jnp.float32),
        pltpu.VMEM((D,), jnp.float32),
        pltpu.SemaphoreType.DMA,
    ],
)
def _sc_embed_ln(xt_hbm, tab_hbm, gamma_hbm, beta_hbm, out_hbm,
                 idx_v, oidx_v, rows_v, g_v, b_v, sem):
    wid = lax.axis_index("s") * NC + lax.axis_index("c")
    base_b = wid * BW

    pltpu.sync_copy(gamma_hbm, g_v)
    pltpu.sync_copy(beta_hbm, b_v)

    pltpu.sync_copy(xt_hbm.at[pl.ds(0, F), pl.ds(base_b, BW)], idx_v)

    iota = lax.iota(jnp.int32, 16)

    def fire(f, carry):
        pltpu.async_copy(
            tab_hbm.at[f].at[idx_v.at[f]],
            rows_v.at[pl.ds(f * BW, BW)],
            sem,
        )
        return carry

    lax.fori_loop(0, F, fire, 0)

    def oidx_fill(f, carry):
        for kk in range(BW // 16):
            b = iota + kk * 16
            oidx_v[f, pl.ds(kk * 16, 16)] = (base_b + b) * F + f
        return carry

    lax.fori_loop(0, F, oidx_fill, 0)

    def drain(f, carry):
        pltpu.make_async_copy(
            tab_hbm.at[f].at[idx_v.at[f]],
            rows_v.at[pl.ds(f * BW, BW)],
            sem,
        ).wait()
        return carry

    lax.fori_loop(0, F, drain, 0)

    def ln_block(bi, carry):
        ids = iota + bi * 16
        vs = []
        acc = jnp.zeros((16,), jnp.float32)
        acc2 = jnp.zeros((16,), jnp.float32)
        for d in range(D):
            col = jnp.full((16,), d, jnp.int32)
            v = plsc.load_gather(rows_v, [ids, col])
            vs.append(v)
            acc = acc + v
            acc2 = acc2 + v * v
        mean = acc * (1.0 / D)
        var = acc2 * (1.0 / D) - mean * mean
        r = _rsqrt(var + EPS)
        for d in range(D):
            col = jnp.full((16,), d, jnp.int32)
            g = plsc.load_gather(g_v, [col])
            b = plsc.load_gather(b_v, [col])
            o = (vs[d] - mean) * r * g + b
            plsc.store_scatter(rows_v, [ids, col], o)
        return carry

    lax.fori_loop(0, NB, ln_block, 0)

    def scat(f, carry):
        pltpu.async_copy(
            rows_v.at[pl.ds(f * BW, BW)],
            out_hbm.at[oidx_v.at[f]],
            sem,
        )
        return carry

    lax.fori_loop(0, F, scat, 0)

    def scat_drain(f, carry):
        pltpu.make_async_copy(
            rows_v.at[pl.ds(f * BW, BW)],
            out_hbm.at[oidx_v.at[f]],
            sem,
        ).wait()
        return carry

    lax.fori_loop(0, F, scat_drain, 0)


def kernel(x, tables, gamma, beta):
    xt = x.T.astype(jnp.int32)
    out = _sc_embed_ln(xt, tables,
                       gamma.astype(jnp.float32), beta.astype(jnp.float32))
    return out.reshape(B, F, D)


# speedup vs baseline: 4.1350x; 4.1350x over previous
import functools

import jax
import jax.numpy as jnp
from jax import lax
from jax.experimental import pallas as pl
from jax.experimental.pallas import tpu as pltpu
from jax.experimental.pallas import tpu_sc as plsc

F = 26
V = 100000
D = 32
B = 4096
EPS = 1e-5

NC = 2
NS = 16
NB16 = B // 16


def _rsqrt(x):
    i = plsc.bitcast(x, jnp.int32)
    i = jnp.int32(0x5F3759DF) - (i >> 1)
    y = plsc.bitcast(i, jnp.float32)
    for _ in range(3):
        y = y * (1.5 - 0.5 * x * y * y)
    return y


@functools.partial(
    pl.kernel,
    out_type=jax.ShapeDtypeStruct((F, D, B), jnp.float32),
    mesh=plsc.VectorSubcoreMesh(
        core_axis_name="c", subcore_axis_name="s", num_cores=NC, num_subcores=NS
    ),
    compiler_params=pltpu.CompilerParams(
        use_tc_tiling_on_sc=True, needs_layout_passes=False
    ),
    scratch_types=[
        pltpu.VMEM((V,), jnp.float32),
        pltpu.VMEM((B,), jnp.int32),
        pltpu.VMEM((B,), jnp.float32),
        pltpu.VMEM((B,), jnp.float32),
        pltpu.VMEM((B,), jnp.float32),
        pltpu.VMEM((D,), jnp.float32),
        pltpu.VMEM((D,), jnp.float32),
        pltpu.SemaphoreType.DMA,
    ],
)
def _sc_embed_ln(xt_hbm, tab_hbm, gamma_hbm, beta_hbm, out_hbm,
                 win_v, xcol_v, col_v, acc_v, acc2_v, g_v, b_v, sem):
    wid = lax.axis_index("s") * NC + lax.axis_index("c")

    @pl.when(wid < F)
    def _work():
        f = wid
        pltpu.sync_copy(gamma_hbm, g_v)
        pltpu.sync_copy(beta_hbm, b_v)
        pltpu.sync_copy(xt_hbm.at[f], xcol_v)

        zeros = jnp.zeros((16,), jnp.float32)

        def zero_acc(j, carry):
            s = pl.ds(j * 16, 16)
            acc_v[s] = zeros
            acc2_v[s] = zeros
            return carry

        lax.fori_loop(0, NB16, zero_acc, 0)

        def pass1(d, carry):
            pltpu.sync_copy(tab_hbm.at[f, d], win_v)

            def inner(j, c):
                s = pl.ds(j * 16, 16)
                w = plsc.load_gather(win_v, [xcol_v[s]])
                acc_v[s] = acc_v[s] + w
                acc2_v[s] = acc2_v[s] + w * w
                col_v[s] = w
                return c

            lax.fori_loop(0, NB16, inner, 0)
            pltpu.sync_copy(col_v, out_hbm.at[f, d])
            return carry

        lax.fori_loop(0, D, pass1, 0)

        def fin(j, carry):
            s = pl.ds(j * 16, 16)
            m = acc_v[s] * (1.0 / D)
            var = acc2_v[s] * (1.0 / D) - m * m
            acc_v[s] = m
            acc2_v[s] = _rsqrt(var + EPS)
            return carry

        lax.fori_loop(0, NB16, fin, 0)

        def pass2(d, carry):
            pltpu.sync_copy(out_hbm.at[f, d], col_v)
            dcol = jnp.zeros((16,), jnp.int32) + d
            g = plsc.load_gather(g_v, [dcol])
            b = plsc.load_gather(b_v, [dcol])

            def inner(j, c):
                s = pl.ds(j * 16, 16)
                col_v[s] = (col_v[s] - acc_v[s]) * acc2_v[s] * g + b
                return c

            lax.fori_loop(0, NB16, inner, 0)
            pltpu.sync_copy(col_v, out_hbm.at[f, d])
            return carry

        lax.fori_loop(0, D, pass2, 0)


def kernel(x, tables, gamma, beta):
    xt = x.T.astype(jnp.int32)
    tab_t = jnp.transpose(tables, (0, 2, 1))
    out = _sc_embed_ln(xt, tab_t,
                       gamma.astype(jnp.float32), beta.astype(jnp.float32))
    return jnp.transpose(out, (2, 0, 1))


# speedup vs baseline: 4.9426x; 1.1953x over previous
import functools

import jax
import jax.numpy as jnp
from jax import lax
from jax.experimental import pallas as pl
from jax.experimental.pallas import tpu as pltpu
from jax.experimental.pallas import tpu_sc as plsc

F = 26
V = 100000
D = 32
B = 4096
EPS = 1e-5

NC = 2
NS = 16
NB16 = B // 16
H0 = 50048
H1 = V - H0


def _rsqrt(x):
    i = plsc.bitcast(x, jnp.int32)
    i = jnp.int32(0x5F3759DF) - (i >> 1)
    y = plsc.bitcast(i, jnp.float32)
    for _ in range(3):
        y = y * (1.5 - 0.5 * x * y * y)
    return y


@functools.partial(
    pl.kernel,
    out_type=jax.ShapeDtypeStruct((F, D, B), jnp.float32),
    mesh=plsc.VectorSubcoreMesh(
        core_axis_name="c", subcore_axis_name="s", num_cores=NC, num_subcores=NS
    ),
    compiler_params=pltpu.CompilerParams(
        use_tc_tiling_on_sc=True, needs_layout_passes=False
    ),
    scratch_types=[
        pltpu.VMEM((H0,), jnp.float32),
        pltpu.VMEM((H1,), jnp.float32),
        pltpu.VMEM((B,), jnp.int32),
        pltpu.VMEM((B,), jnp.float32),
        pltpu.VMEM((B,), jnp.float32),
        pltpu.VMEM((B,), jnp.float32),
        pltpu.VMEM((D,), jnp.float32),
        pltpu.VMEM((D,), jnp.float32),
        pltpu.SemaphoreType.DMA,
        pltpu.SemaphoreType.DMA,
    ],
)
def _sc_embed_ln(xt_hbm, tab_hbm, gamma_hbm, beta_hbm, out_hbm,
                 win0_v, win1_v, xcol_v, col_v, acc_v, acc2_v, g_v, b_v,
                 sem0, sem1):
    wid = lax.axis_index("s") * NC + lax.axis_index("c")

    @pl.when(wid < F)
    def _work():
        f = wid
        pltpu.sync_copy(gamma_hbm, g_v)
        pltpu.sync_copy(beta_hbm, b_v)
        pltpu.sync_copy(xt_hbm.at[f], xcol_v)

        zeros = jnp.zeros((16,), jnp.float32)

        def zero_acc(j, carry):
            s = pl.ds(j * 16, 16)
            acc_v[s] = zeros
            acc2_v[s] = zeros
            return carry

        lax.fori_loop(0, NB16, zero_acc, 0)

        def fire0(d):
            pltpu.async_copy(tab_hbm.at[f, d, pl.ds(0, H0)], win0_v, sem0)

        def fire1(d):
            pltpu.async_copy(tab_hbm.at[f, d, pl.ds(H0, H1)], win1_v, sem1)

        def wait0(d):
            pltpu.make_async_copy(
                tab_hbm.at[f, d, pl.ds(0, H0)], win0_v, sem0
            ).wait()

        def wait1(d):
            pltpu.make_async_copy(
                tab_hbm.at[f, d, pl.ds(H0, H1)], win1_v, sem1
            ).wait()

        fire0(0)
        fire1(0)

        def pass1(d, carry):
            wait0(d)

            def inner0(j, c):
                s = pl.ds(j * 16, 16)
                idx = xcol_v[s]
                m = idx < H0
                w = jnp.where(m, plsc.load_gather(win0_v, [idx], mask=m), 0.0)
                acc_v[s] = acc_v[s] + w
                acc2_v[s] = acc2_v[s] + w * w
                col_v[s] = w
                return c

            lax.fori_loop(0, NB16, inner0, 0)

            @pl.when(d < D - 1)
            def _prefetch0():
                fire0(d + 1)

            wait1(d)

            def inner1(j, c):
                s = pl.ds(j * 16, 16)
                idx = xcol_v[s]
                m = idx >= H0
                w = jnp.where(
                    m, plsc.load_gather(win1_v, [idx - H0], mask=m), 0.0
                )
                acc_v[s] = acc_v[s] + w
                acc2_v[s] = acc2_v[s] + w * w
                col_v[s] = col_v[s] + w
                return c

            lax.fori_loop(0, NB16, inner1, 0)

            @pl.when(d < D - 1)
            def _prefetch1():
                fire1(d + 1)

            pltpu.sync_copy(col_v, out_hbm.at[f, d])
            return carry

        lax.fori_loop(0, D, pass1, 0)

        def fin(j, carry):
            s = pl.ds(j * 16, 16)
            m = acc_v[s] * (1.0 / D)
            var = acc2_v[s] * (1.0 / D) - m * m
            acc_v[s] = m
            acc2_v[s] = _rsqrt(var + EPS)
            return carry

        lax.fori_loop(0, NB16, fin, 0)

        def pass2(d, carry):
            pltpu.sync_copy(out_hbm.at[f, d], col_v)
            dcol = jnp.zeros((16,), jnp.int32) + d
            g = plsc.load_gather(g_v, [dcol])
            b = plsc.load_gather(b_v, [dcol])

            def inner(j, c):
                s = pl.ds(j * 16, 16)
                col_v[s] = (col_v[s] - acc_v[s]) * acc2_v[s] * g + b
                return c

            lax.fori_loop(0, NB16, inner, 0)
            pltpu.sync_copy(col_v, out_hbm.at[f, d])
            return carry

        lax.fori_loop(0, D, pass2, 0)


def kernel(x, tables, gamma, beta):
    xt = x.T.astype(jnp.int32)
    tab_t = jnp.transpose(tables, (0, 2, 1))
    out = _sc_embed_ln(xt, tab_t,
                       gamma.astype(jnp.float32), beta.astype(jnp.float32))
    return jnp.transpose(out, (2, 0, 1))


# speedup vs baseline: 4.9575x; 1.0030x over previous
import functools

import jax
import jax.numpy as jnp
from jax import lax
from jax.experimental import pallas as pl
from jax.experimental.pallas import tpu as pltpu
from jax.experimental.pallas import tpu_sc as plsc

F = 26
V = 100000
D = 32
B = 4096
EPS = 1e-5

NC = 2
NS = 16
NB16 = B // 16
H0 = 50048
H1 = V - H0


def _rsqrt(x):
    i = plsc.bitcast(x, jnp.int32)
    i = jnp.int32(0x5F3759DF) - (i >> 1)
    y = plsc.bitcast(i, jnp.float32)
    for _ in range(3):
        y = y * (1.5 - 0.5 * x * y * y)
    return y


@functools.partial(
    pl.kernel,
    out_type=jax.ShapeDtypeStruct((F, D, B), jnp.float32),
    mesh=plsc.VectorSubcoreMesh(
        core_axis_name="c", subcore_axis_name="s", num_cores=NC, num_subcores=NS
    ),
    compiler_params=pltpu.CompilerParams(
        use_tc_tiling_on_sc=True, needs_layout_passes=False
    ),
    scratch_types=[
        pltpu.VMEM((H0,), jnp.float32),
        pltpu.VMEM((H1,), jnp.float32),
        pltpu.VMEM((B,), jnp.int32),
        pltpu.VMEM((B,), jnp.float32),
        pltpu.VMEM((B,), jnp.float32),
        pltpu.VMEM((B,), jnp.float32),
        pltpu.VMEM((D,), jnp.float32),
        pltpu.VMEM((D,), jnp.float32),
        pltpu.SemaphoreType.DMA,
        pltpu.SemaphoreType.DMA,
    ],
)
def _sc_embed_ln(xt_hbm, tab_hbm, gamma_hbm, beta_hbm, out_hbm,
                 win0_v, win1_v, xcol_v, col_v, acc_v, acc2_v, g_v, b_v,
                 sem0, sem1):
    wid = lax.axis_index("s") * NC + lax.axis_index("c")

    @pl.when(wid < F)
    def _work():
        f = wid
        pltpu.sync_copy(gamma_hbm, g_v)
        pltpu.sync_copy(beta_hbm, b_v)
        pltpu.sync_copy(xt_hbm.at[f], xcol_v)

        zeros = jnp.zeros((16,), jnp.float32)

        def zero_acc(j, carry):
            s = pl.ds(j * 16, 16)
            acc_v[s] = zeros
            acc2_v[s] = zeros
            return carry

        lax.fori_loop(0, NB16, zero_acc, 0)

        def fire0(d):
            pltpu.async_copy(tab_hbm.at[f, d, pl.ds(0, H0)], win0_v, sem0)

        def fire1(d):
            pltpu.async_copy(tab_hbm.at[f, d, pl.ds(H0, H1)], win1_v, sem1)

        def wait0(d):
            pltpu.make_async_copy(
                tab_hbm.at[f, d, pl.ds(0, H0)], win0_v, sem0
            ).wait()

        def wait1(d):
            pltpu.make_async_copy(
                tab_hbm.at[f, d, pl.ds(H0, H1)], win1_v, sem1
            ).wait()

        fire0(0)
        fire1(0)

        def pass1(d, carry):
            wait0(d)

            def inner0(j, c):
                for u in range(4):
                    s = pl.ds(j * 64 + u * 16, 16)
                    idx = xcol_v[s]
                    m = idx < H0
                    w = jnp.where(
                        m, plsc.load_gather(win0_v, [idx], mask=m), 0.0
                    )
                    acc_v[s] = acc_v[s] + w
                    acc2_v[s] = acc2_v[s] + w * w
                    col_v[s] = w
                return c

            lax.fori_loop(0, NB16 // 4, inner0, 0)

            @pl.when(d < D - 1)
            def _prefetch0():
                fire0(d + 1)

            wait1(d)

            def inner1(j, c):
                for u in range(4):
                    s = pl.ds(j * 64 + u * 16, 16)
                    idx = xcol_v[s]
                    m = idx >= H0
                    w = jnp.where(
                        m, plsc.load_gather(win1_v, [idx - H0], mask=m), 0.0
                    )
                    acc_v[s] = acc_v[s] + w
                    acc2_v[s] = acc2_v[s] + w * w
                    col_v[s] = col_v[s] + w
                return c

            lax.fori_loop(0, NB16 // 4, inner1, 0)

            @pl.when(d < D - 1)
            def _prefetch1():
                fire1(d + 1)

            pltpu.sync_copy(col_v, out_hbm.at[f, d])
            return carry

        lax.fori_loop(0, D, pass1, 0)

        def fin(j, carry):
            s = pl.ds(j * 16, 16)
            m = acc_v[s] * (1.0 / D)
            var = acc2_v[s] * (1.0 / D) - m * m
            acc_v[s] = m
            acc2_v[s] = _rsqrt(var + EPS)
            return carry

        lax.fori_loop(0, NB16, fin, 0)

        def pass2(d, carry):
            pltpu.sync_copy(out_hbm.at[f, d], col_v)
            dcol = jnp.zeros((16,), jnp.int32) + d
            g = plsc.load_gather(g_v, [dcol])
            b = plsc.load_gather(b_v, [dcol])

            def inner(j, c):
                s = pl.ds(j * 16, 16)
                col_v[s] = (col_v[s] - acc_v[s]) * acc2_v[s] * g + b
                return c

            lax.fori_loop(0, NB16, inner, 0)
            pltpu.sync_copy(col_v, out_hbm.at[f, d])
            return carry

        lax.fori_loop(0, D, pass2, 0)


def kernel(x, tables, gamma, beta):
    xt = x.T.astype(jnp.int32)
    tab_t = jnp.transpose(tables, (0, 2, 1))
    out = _sc_embed_ln(xt, tab_t,
                       gamma.astype(jnp.float32), beta.astype(jnp.float32))
    return jnp.transpose(out, (2, 0, 1))


# speedup vs baseline: 6.4837x; 1.3078x over previous
import functools

import jax
import jax.numpy as jnp
from jax import lax
from jax.experimental import pallas as pl
from jax.experimental.pallas import tpu as pltpu
from jax.experimental.pallas import tpu_sc as plsc

F = 26
V = 100000
D = 32
B = 4096
EPS = 1e-5

NC = 2
NS = 16
NB16 = B // 16
H0 = 50048
H1 = V - H0


def _rsqrt(x):
    i = plsc.bitcast(x, jnp.int32)
    i = jnp.int32(0x5F3759DF) - (i >> 1)
    y = plsc.bitcast(i, jnp.float32)
    for _ in range(3):
        y = y * (1.5 - 0.5 * x * y * y)
    return y


@functools.partial(
    pl.kernel,
    out_type=jax.ShapeDtypeStruct((F, D, B), jnp.float32),
    mesh=plsc.VectorSubcoreMesh(
        core_axis_name="c", subcore_axis_name="s", num_cores=NC, num_subcores=NS
    ),
    compiler_params=pltpu.CompilerParams(
        use_tc_tiling_on_sc=True, needs_layout_passes=False
    ),
    scratch_types=[
        pltpu.VMEM((H0,), jnp.float32),
        pltpu.VMEM((H1,), jnp.float32),
        pltpu.VMEM((B,), jnp.int32),
        pltpu.VMEM((B,), jnp.float32),
        pltpu.VMEM((B,), jnp.float32),
        pltpu.VMEM((B,), jnp.float32),
        pltpu.VMEM((D,), jnp.float32),
        pltpu.VMEM((D,), jnp.float32),
        pltpu.SemaphoreType.DMA,
        pltpu.SemaphoreType.DMA,
    ],
)
def _sc_embed_ln(xt_hbm, tab_hbm, gamma_hbm, beta_hbm, out_hbm,
                 win0_v, win1_v, xcol_v, col_v, acc_v, acc2_v, g_v, b_v,
                 sem0, sem1):
    wid = lax.axis_index("s") * NC + lax.axis_index("c")

    @pl.when(wid < F)
    def _work():
        f = wid
        pltpu.sync_copy(gamma_hbm, g_v)
        pltpu.sync_copy(beta_hbm, b_v)
        pltpu.sync_copy(xt_hbm.at[f], xcol_v)

        zeros = jnp.zeros((16,), jnp.float32)

        def zero_acc(j, carry):
            s = pl.ds(j * 16, 16)
            acc_v[s] = zeros
            acc2_v[s] = zeros
            return carry

        lax.fori_loop(0, NB16, zero_acc, 0)

        def fire0(d):
            pltpu.async_copy(tab_hbm.at[f, d, pl.ds(0, H0)], win0_v, sem0)

        def fire1(d):
            pltpu.async_copy(tab_hbm.at[f, d, pl.ds(H0, H1)], win1_v, sem1)

        def wait0(d):
            pltpu.make_async_copy(
                tab_hbm.at[f, d, pl.ds(0, H0)], win0_v, sem0
            ).wait()

        def wait1(d):
            pltpu.make_async_copy(
                tab_hbm.at[f, d, pl.ds(H0, H1)], win1_v, sem1
            ).wait()

        fire0(0)
        fire1(0)

        def pass1(d, carry):
            wait0(d)

            def inner0(j, c):
                for u in range(4):
                    s = pl.ds(j * 64 + u * 16, 16)
                    idx = xcol_v[s]
                    m = idx < H0
                    w = jnp.where(
                        m, plsc.load_gather(win0_v, [idx], mask=m), 0.0
                    )
                    acc_v[s] = acc_v[s] + w
                    acc2_v[s] = acc2_v[s] + w * w
                    col_v[s] = w
                return c

            lax.fori_loop(0, NB16 // 4, inner0, 0)

            @pl.when(d < D - 1)
            def _prefetch0():
                fire0(d + 1)

            wait1(d)

            def inner1(j, c):
                for u in range(4):
                    s = pl.ds(j * 64 + u * 16, 16)
                    idx = xcol_v[s]
                    m = idx >= H0
                    w = jnp.where(
                        m, plsc.load_gather(win1_v, [idx - H0], mask=m), 0.0
                    )
                    acc_v[s] = acc_v[s] + w
                    acc2_v[s] = acc2_v[s] + w * w
                    col_v[s] = col_v[s] + w
                return c

            lax.fori_loop(0, NB16 // 4, inner1, 0)

            @pl.when(d < D - 1)
            def _prefetch1():
                fire1(d + 1)

            pltpu.sync_copy(col_v, out_hbm.at[f, d])
            return carry

        lax.fori_loop(0, D, pass1, 0)

        def fin(j, carry):
            s = pl.ds(j * 16, 16)
            m = acc_v[s] * (1.0 / D)
            var = acc2_v[s] * (1.0 / D) - m * m
            acc_v[s] = m
            acc2_v[s] = _rsqrt(var + EPS)
            return carry

        lax.fori_loop(0, NB16, fin, 0)

        colA = win0_v.at[pl.ds(0, B)]
        colB = win1_v.at[pl.ds(0, B)]

        def rd(d, buf, sem):
            pltpu.async_copy(out_hbm.at[f, d], buf, sem)

        def wr(d, buf, sem):
            pltpu.async_copy(buf, out_hbm.at[f, d], sem)

        def wt(buf, sem):
            pltpu.make_async_copy(out_hbm.at[f, 0], buf, sem).wait()

        def normalize(buf, d):
            dcol = jnp.zeros((16,), jnp.int32) + d
            g = plsc.load_gather(g_v, [dcol])
            b = plsc.load_gather(b_v, [dcol])

            def inner(j, c):
                for u in range(4):
                    s = pl.ds(j * 64 + u * 16, 16)
                    buf[s] = (buf[s] - acc_v[s]) * acc2_v[s] * g + b
                return c

            lax.fori_loop(0, NB16 // 4, inner, 0)

        rd(0, colA, sem0)
        rd(1, colB, sem1)

        def pass2(i, carry):
            d0 = i * 2
            wt(colA, sem0)
            normalize(colA, d0)
            wr(d0, colA, sem0)
            wt(colB, sem1)
            normalize(colB, d0 + 1)
            wr(d0 + 1, colB, sem1)

            @pl.when(i < D // 2 - 1)
            def _next():
                wt(colA, sem0)
                rd(d0 + 2, colA, sem0)
                wt(colB, sem1)
                rd(d0 + 3, colB, sem1)

            return carry

        lax.fori_loop(0, D // 2, pass2, 0)
        wt(colA, sem0)
        wt(colB, sem1)


def kernel(x, tables, gamma, beta):
    xt = x.T.astype(jnp.int32)
    tab_t = jnp.transpose(tables, (0, 2, 1))
    out = _sc_embed_ln(xt, tab_t,
                       gamma.astype(jnp.float32), beta.astype(jnp.float32))
    return jnp.transpose(out, (2, 0, 1))
